# Initial kernel scaffold; baseline (speedup 1.0000x reference)
import jax
import jax.numpy as jnp
from jax.experimental import pallas as pl


def kernel(x, edge_index, edge_attr, batch, Wl, bl, Wr, br, We, att, bias_out, Wp, bp):
    raise NotImplementedError("write your pallas kernel here")


# speedup vs baseline: 20.2385x; 20.2385x over previous
import functools

import jax
import jax.numpy as jnp
from jax import lax
from jax.experimental import pallas as pl
from jax.experimental.pallas import tpu as pltpu
from jax.experimental.pallas import tpu_sc as plsc

N = 10000
E = 320000
D = 128
NH = 4
CH = 32
G = 64

NC = 2
NS = 16
NW = NC * NS
EPT = E // NW
K = 40
NBLK = EPT // K
NPT = N // NS
NJ = D // 16


def _proj_body(x_ref, wl_ref, bl_ref, wr_ref, br_ref, we_ref, att_ref,
               xl_ref, xr_ref, ct_ref):
    x = x_ref[...]
    dn = (((1,), (1,)), ((), ()))
    xl = lax.dot_general(x, wl_ref[...], dn,
                         preferred_element_type=jnp.float32) + bl_ref[...]
    xr = lax.dot_general(x, wr_ref[...], dn,
                         preferred_element_type=jnp.float32) + br_ref[...]
    xl_ref[...] = xl
    xr_ref[...] = xr
    g = xl + xr + we_ref[...]
    g = jnp.maximum(g, 0.2 * g)
    ga = g * att_ref[...]
    blk = x.shape[0]
    cols = [jnp.broadcast_to(
        jnp.sum(ga[:, h * CH:(h + 1) * CH], axis=1, keepdims=True), (blk, CH))
        for h in range(NH)]
    ct_ref[...] = jnp.concatenate(cols, axis=1)


def _phase0(x, wl, bl, wr, br, we_row, att_row):
    blk = 2000
    nblk = N // blk
    full = pl.BlockSpec((D, D), lambda b: (0, 0))
    row = pl.BlockSpec((1, D), lambda b: (0, 0))
    return pl.pallas_call(
        _proj_body,
        grid=(nblk,),
        in_specs=[pl.BlockSpec((blk, D), lambda b: (b, 0)),
                  full, row, full, row, row, row],
        out_specs=[pl.BlockSpec((blk, D), lambda b: (b, 0)),
                   pl.BlockSpec((blk, D), lambda b: (b, 0)),
                   pl.BlockSpec((blk, D), lambda b: (b, 0))],
        out_shape=[jax.ShapeDtypeStruct((N, D), jnp.float32),
                   jax.ShapeDtypeStruct((N, D), jnp.float32),
                   jax.ShapeDtypeStruct((N, D), jnp.float32)],
    )(x, wl, bl, wr, br, we_row, att_row)


def _bcast_body(ea_ref, dm_ref, out_ref):
    blk = ea_ref.shape[0]
    out_ref[...] = jnp.concatenate(
        [jnp.broadcast_to(ea_ref[...], (blk, 16)),
         jnp.broadcast_to(dm_ref[...], (blk, 16))], axis=1)


def _phase0b(ea2, dm2):
    blk = 4000
    nblk = E // blk
    return pl.pallas_call(
        _bcast_body,
        grid=(nblk,),
        in_specs=[pl.BlockSpec((blk, 1), lambda b: (b, 0)),
                  pl.BlockSpec((blk, 1), lambda b: (b, 0))],
        out_specs=pl.BlockSpec((blk, 32), lambda b: (b, 0)),
        out_shape=jax.ShapeDtypeStruct((E, 32), jnp.float32),
    )(ea2, dm2)


DEN2R = 1250


def _sc_body(xl, xr, ct, srce, dste, dst8e, eadm, we, att, acc_out, den_out,
             sidx, didx, didx8, eadmrows, rows_l, rows_r, crows, msgbuf,
             denbuf, zidx, sumbuf, we_v, att_v, acc_sh, den_sh,
             s1, s2, s3, s4):
    c = lax.axis_index("c")
    s = lax.axis_index("s")
    wid = s * NC + c
    iota16 = lax.broadcasted_iota(jnp.int32, (16,), 0)
    fzero = jnp.zeros((16,), jnp.float32)

    def zbuf(e, _):
        for j in range(NJ):
            msgbuf[e, pl.ds(j * 16, 16)] = fzero
            denbuf[e, pl.ds(j * 16, 16)] = fzero
        return 0

    lax.fori_loop(0, K, zbuf, 0)

    base_node = s * NPT
    for shot in range(16):
        for off in (0, 16, K - 16):
            v = jnp.minimum(jnp.full((16,), base_node + shot * K + off,
                                     jnp.int32) + iota16, N - 1)
            zidx[pl.ds(off, 16)] = v
        pltpu.sync_copy(msgbuf, acc_sh.at[zidx])
    for shot in range(2):
        for off in (0, 16, K - 16):
            v = jnp.minimum(jnp.full((16,), s * 2 * K + shot * K + off,
                                     jnp.int32) + iota16, DEN2R - 1)
            zidx[pl.ds(off, 16)] = v
        pltpu.sync_copy(denbuf, den_sh.at[zidx])

    pltpu.sync_copy(we, we_v)
    pltpu.sync_copy(att, att_v)
    plsc.subcore_barrier()

    def blk_body(b, carry):
        base = wid * EPT + b * K
        pltpu.sync_copy(srce.at[pl.ds(base, K)], sidx)
        pltpu.sync_copy(dste.at[pl.ds(base, K)], didx)
        pltpu.sync_copy(dst8e.at[pl.ds(base, K)], didx8)
        d4 = pltpu.async_copy(eadm.at[pl.ds(base, K)], eadmrows, s4)
        d1 = pltpu.async_copy(xl.at[sidx], rows_l, s1)
        d2 = pltpu.async_copy(xr.at[didx], rows_r, s2)
        d3 = pltpu.async_copy(ct.at[didx], crows, s3)
        d1.wait()
        d2.wait()
        d3.wait()
        d4.wait()

        def edge_body(e, _):
            eav = eadmrows[e, pl.ds(0, 16)]
            dmv = eadmrows[e, pl.ds(16, 16)]
            gl = []
            ha = []
            for j in range(NJ):
                glj = rows_l[e, pl.ds(j * 16, 16)]
                grj = rows_r[e, pl.ds(j * 16, 16)]
                gv = glj + grj + eav * we_v[pl.ds(j * 16, 16)]
                gv = jnp.maximum(gv, 0.2 * gv)
                gl.append(glj)
                ha.append(gv * att_v[pl.ds(j * 16, 16)])
            pv = []
            for h in range(NH):
                v = ha[2 * h] + ha[2 * h + 1]
                for r in (8, 4, 2, 1):
                    sumbuf[pl.ds(32 * h, 16)] = v
                    sumbuf[pl.ds(32 * h + 16, 16)] = v
                    v = v + sumbuf[pl.ds(32 * h + r, 16)]
                cb = crows[e, pl.ds(CH * h, 16)]
                pv.append(jnp.exp(v - cb))
            pden = fzero
            for h in range(NH):
                pden = pden + jnp.where(iota16 == h, pv[h], 0.0)
            for slot in range(8):
                denbuf[e, pl.ds(slot * 16, 16)] = jnp.where(
                    dmv == jnp.float32(slot), pden, 0.0)
            for j in range(NJ):
                msgbuf[e, pl.ds(j * 16, 16)] = gl[j] * pv[j // 2]
            return 0

        lax.fori_loop(0, K, edge_body, 0)
        pltpu.sync_copy(msgbuf, acc_sh.at[didx], add=True)
        pltpu.sync_copy(denbuf, den_sh.at[didx8], add=True)
        return carry

    lax.fori_loop(0, NBLK, blk_body, 0)
    plsc.subcore_barrier()

    wr = 624
    pltpu.sync_copy(acc_sh.at[pl.ds(s * wr, wr)],
                    acc_out.at[c, pl.ds(s * wr, wr)])

    @pl.when(s == NS - 1)
    def _():
        tail = NS * wr
        pltpu.sync_copy(acc_sh.at[pl.ds(tail, N - tail)],
                        acc_out.at[c, pl.ds(tail, N - tail)])

    @pl.when(s < NS - 1)
    def _():
        pltpu.sync_copy(den_sh.at[pl.ds(s * K, K)],
                        den_out.at[c, pl.ds(s * K, K)])

    @pl.when(s == NS - 1)
    def _():
        dtail = (NS - 1) * K
        pltpu.sync_copy(den_sh.at[pl.ds(dtail, DEN2R - dtail)],
                        den_out.at[c, pl.ds(dtail, DEN2R - dtail)])


def _phase1(xl, xr, ct, src, dst, dst8, eadm, we_col, att_col):
    mesh = plsc.VectorSubcoreMesh(core_axis_name="c", subcore_axis_name="s")
    fn = functools.partial(
        pl.kernel,
        out_type=(jax.ShapeDtypeStruct((NC, N, D), jnp.float32),
                  jax.ShapeDtypeStruct((NC, DEN2R, D), jnp.float32)),
        mesh=mesh,
        scratch_types=[
            pltpu.VMEM((K,), jnp.int32),
            pltpu.VMEM((K,), jnp.int32),
            pltpu.VMEM((K,), jnp.int32),
            pltpu.VMEM((K, 32), jnp.float32),
            pltpu.VMEM((K, D), jnp.float32),
            pltpu.VMEM((K, D), jnp.float32),
            pltpu.VMEM((K, D), jnp.float32),
            pltpu.VMEM((K, D), jnp.float32),
            pltpu.VMEM((K, D), jnp.float32),
            pltpu.VMEM((K,), jnp.int32),
            pltpu.VMEM((32 * NH,), jnp.float32),
            pltpu.VMEM((D,), jnp.float32),
            pltpu.VMEM((D,), jnp.float32),
            pltpu.VMEM_SHARED((N, D), jnp.float32),
            pltpu.VMEM_SHARED((DEN2R, D), jnp.float32),
            pltpu.SemaphoreType.DMA,
            pltpu.SemaphoreType.DMA,
            pltpu.SemaphoreType.DMA,
            pltpu.SemaphoreType.DMA,
        ],
    )(_sc_body)
    return fn(xl, xr, ct, src, dst, dst8, eadm, we_col, att_col)


def _pool_body(acc0_ref, acc1_ref, den0_ref, den1_ref, xl_ref, bt_ref,
               bias_ref, wp_ref, bp_ref, out_ref, psum, cmat):
    b = pl.program_id(0)
    nb = pl.num_programs(0)

    @pl.when(b == 0)
    def _():
        psum[...] = jnp.zeros_like(psum)
        cmat[...] = jnp.zeros_like(cmat)

    a = acc0_ref[...] + acc1_ref[...] + xl_ref[...]
    d = den0_ref[...] + den1_ref[...]
    cols = [a[:, h * CH:(h + 1) * CH] / (d[:, h:h + 1] + 1.0 + 1e-16)
            for h in range(NH)]
    outb = jnp.concatenate(cols, axis=1)

    blk = outb.shape[0]
    bt = bt_ref[...]
    e = (bt == lax.broadcasted_iota(jnp.int32, (blk, G), 1)).astype(jnp.float32)
    dn = (((0,), (0,)), ((), ()))
    psum[...] += lax.dot_general(e, outb, dn,
                                 preferred_element_type=jnp.float32)
    cmat[...] += lax.dot_general(e, jnp.ones((blk, D), jnp.float32), dn,
                                 preferred_element_type=jnp.float32)

    @pl.when(b == nb - 1)
    def _():
        cm = cmat[...]
        pooled = psum[...] / jnp.maximum(cm, 1.0)
        pooled = pooled + jnp.where(cm > 0.0, bias_ref[...], 0.0)
        r = jnp.sum(pooled * wp_ref[...], axis=1, keepdims=True) + bp_ref[...]
        out_ref[...] = r


def _phase2(acc0, acc1, den0, den1, xl, batch2, bias_row, wp_row, bp11):
    blk = 2000
    nblk = N // blk
    row = pl.BlockSpec((1, D), lambda b: (0, 0))
    return pl.pallas_call(
        _pool_body,
        grid=(nblk,),
        in_specs=[pl.BlockSpec((blk, D), lambda b: (b, 0)),
                  pl.BlockSpec((blk, D), lambda b: (b, 0)),
                  pl.BlockSpec((blk, 16), lambda b: (b, 0)),
                  pl.BlockSpec((blk, 16), lambda b: (b, 0)),
                  pl.BlockSpec((blk, D), lambda b: (b, 0)),
                  pl.BlockSpec((blk, 1), lambda b: (b, 0)),
                  row, row, pl.BlockSpec((1, 1), lambda b: (0, 0))],
        out_specs=pl.BlockSpec((G, 1), lambda b: (0, 0)),
        out_shape=jax.ShapeDtypeStruct((G, 1), jnp.float32),
        scratch_shapes=[pltpu.VMEM((G, D), jnp.float32),
                        pltpu.VMEM((G, D), jnp.float32)],
    )(acc0, acc1, den0, den1, xl, batch2, bias_row, wp_row, bp11)


def kernel(x, edge_index, edge_attr, batch, Wl, bl, Wr, br, We, att,
           bias_out, Wp, bp):
    ei = edge_index.astype(jnp.int32)
    src = ei[0]
    dst = ei[1]
    ea2 = edge_attr.astype(jnp.float32).reshape(E, 1)
    batch2 = batch.astype(jnp.int32).reshape(N, 1)
    we_row = We.reshape(1, D)
    att_row = att.reshape(1, D)
    bl2 = bl.reshape(1, D)
    br2 = br.reshape(1, D)
    bias_row = bias_out.reshape(1, D)
    wp_row = Wp.reshape(1, D)
    bp11 = bp.reshape(1, 1)

    dst8 = lax.shift_right_logical(dst, 3)
    dm2 = jnp.bitwise_and(dst, 7).astype(jnp.float32).reshape(E, 1)

    xl, xr, ct = _phase0(x, Wl, bl2, Wr, br2, we_row, att_row)
    eadm = _phase0b(ea2, dm2)
    acc, den = _phase1(xl, xr, ct, src, dst, dst8, eadm,
                       We.reshape(D), att.reshape(D))
    den0 = den[0].reshape(N, 16)
    den1 = den[1].reshape(N, 16)
    return _phase2(acc[0], acc[1], den0, den1, xl, batch2,
                   bias_row, wp_row, bp11)
